# Initial kernel scaffold; baseline (speedup 1.0000x reference)
import functools

import jax
import jax.numpy as jnp
from jax import lax
from jax.experimental import pallas as pl
from jax.experimental.pallas import tpu as pltpu
from jax.experimental.pallas import tpu_sc as plsc

N = 20000
E = 60000
D = 16
S = 2000
G = 64
H = 128
EA = 5

NC = 2
NS = 16
NW = NC * NS
CH = 118
NCH = 16
EPW = NCH * CH
EP = NW * EPW

NPW = N // NW
PCH = 125
PNCH = 5
NPT = N // NS
SPT = S // NS
PW = 80

TE = 256
TN = 256


def _mesh():
    return plsc.VectorSubcoreMesh(core_axis_name="c", subcore_axis_name="s")


def _make_gather(d):
    @functools.partial(
        pl.kernel,
        mesh=_mesh(),
        out_type=jax.ShapeDtypeStruct((EP, d), jnp.float32),
        scratch_types=[
            pltpu.VMEM((NCH, CH), jnp.int32),
            pltpu.VMEM((EPW, d), jnp.float32),
            pltpu.SemaphoreType.DMA,
        ],
    )
    def gather_k(table, idx, out, idx_v, rows_v, sem):
        wid = lax.axis_index("c") * NS + lax.axis_index("s")
        pltpu.sync_copy(idx.at[wid], idx_v)
        copies = [
            pltpu.async_copy(
                table.at[idx_v.at[j]], rows_v.at[pl.ds(j * CH, CH)], sem
            )
            for j in range(NCH)
        ]
        for cp in copies:
            cp.wait()
        pltpu.sync_copy(rows_v, out.at[pl.ds(wid * EPW, EPW)])

    return gather_k


def _make_scatter(d):
    @functools.partial(
        pl.kernel,
        mesh=_mesh(),
        out_type=jax.ShapeDtypeStruct((NC * N, d), jnp.float32),
        scratch_types=[
            pltpu.VMEM((NCH, CH), jnp.int32),
            pltpu.VMEM((EPW, d), jnp.float32),
            pltpu.VMEM_SHARED((N, d), jnp.float32),
            pltpu.SemaphoreType.DMA,
        ],
    )
    def scatter_k(zeros, msg, idx, out, idx_v, msg_v, acc, sem):
        c = lax.axis_index("c")
        s = lax.axis_index("s")
        wid = c * NS + s
        pltpu.sync_copy(zeros.at[pl.ds(s * NPT, NPT)], acc.at[pl.ds(s * NPT, NPT)])
        pltpu.sync_copy(idx.at[wid], idx_v)
        pltpu.sync_copy(msg.at[pl.ds(wid * EPW, EPW)], msg_v)
        plsc.subcore_barrier()
        for j in range(NCH):
            pltpu.sync_copy(
                msg_v.at[pl.ds(j * CH, CH)], acc.at[idx_v.at[j]], add=True
            )
        plsc.subcore_barrier()
        pltpu.sync_copy(
            acc.at[pl.ds(s * NPT, NPT)], out.at[pl.ds(c * N + s * NPT, NPT)]
        )

    return scatter_k


@functools.partial(
    pl.kernel,
    mesh=_mesh(),
    out_type=jax.ShapeDtypeStruct((NC * S, PW), jnp.float32),
    scratch_types=[
        pltpu.VMEM((PNCH, PCH), jnp.int32),
        pltpu.VMEM((NPW, PW), jnp.float32),
        pltpu.VMEM_SHARED((S, PW), jnp.float32),
        pltpu.SemaphoreType.DMA,
    ],
)
def _pool_k(zeros, rows, idx, out, idx_v, rows_v, acc, sem):
    c = lax.axis_index("c")
    s = lax.axis_index("s")
    wid = c * NS + s
    pltpu.sync_copy(zeros.at[pl.ds(s * SPT, SPT)], acc.at[pl.ds(s * SPT, SPT)])
    pltpu.sync_copy(idx.at[wid], idx_v)
    pltpu.sync_copy(rows.at[pl.ds(wid * NPW, NPW)], rows_v)
    plsc.subcore_barrier()
    for j in range(PNCH):
        pltpu.sync_copy(
            rows_v.at[pl.ds(j * PCH, PCH)], acc.at[idx_v.at[j]], add=True
        )
    plsc.subcore_barrier()
    pltpu.sync_copy(
        acc.at[pl.ds(s * SPT, SPT)], out.at[pl.ds(c * S + s * SPT, SPT)]
    )


def _msg_body(ea_ref, xs_ref, w1_ref, b1_ref, w2_ref, b2r_ref, out_ref,
              *, m_in, m_out):
    i = pl.program_id(0)
    ea = ea_ref[...]
    h = jnp.maximum(
        jnp.dot(ea, w1_ref[...], preferred_element_type=jnp.float32)
        + b1_ref[...],
        0.0,
    )
    wt = jnp.dot(h.astype(jnp.bfloat16), w2_ref[...],
                 preferred_element_type=jnp.float32)
    xs = xs_ref[...]
    acc = jnp.dot(xs, b2r_ref[...], preferred_element_type=jnp.float32)
    for k in range(m_in):
        acc = acc + xs[:, k:k + 1] * wt[:, k * m_out:(k + 1) * m_out]
    row = i * TE + lax.broadcasted_iota(jnp.int32, (TE, 1), 0)
    out_ref[...] = jnp.where(row < E, acc, 0.0)


def _msg_call(ea_p, xs, w1, b1r, w2b, b2r, m_in, m_out):
    grid = (EP // TE,)
    return pl.pallas_call(
        functools.partial(_msg_body, m_in=m_in, m_out=m_out),
        grid=grid,
        in_specs=[
            pl.BlockSpec((TE, EA), lambda i: (i, 0)),
            pl.BlockSpec((TE, m_in), lambda i: (i, 0)),
            pl.BlockSpec((EA, H), lambda i: (0, 0)),
            pl.BlockSpec((1, H), lambda i: (0, 0)),
            pl.BlockSpec((H, m_in * m_out), lambda i: (0, 0)),
            pl.BlockSpec((m_in, m_out), lambda i: (0, 0)),
        ],
        out_specs=pl.BlockSpec((TE, m_out), lambda i: (i, 0)),
        out_shape=jax.ShapeDtypeStruct((EP, m_out), jnp.float32),
    )(ea_p, xs, w1, b1r, w2b, b2r)


def _elu(y):
    return jnp.where(y > 0, y, jnp.expm1(y))


def _upd_body(p0_ref, p1_ref, x_ref, root_ref, bias_ref, out_ref,
              *, append_ones):
    y = (p0_ref[...] + p1_ref[...]
         + jnp.dot(x_ref[...], root_ref[...],
                   preferred_element_type=jnp.float32)
         + bias_ref[...])
    hout = _elu(y)
    if append_ones:
        out_ref[...] = jnp.concatenate(
            [hout, jnp.ones((hout.shape[0], PW - 64), jnp.float32)], axis=1)
    else:
        out_ref[...] = hout


def _upd_call(p0, p1, x, root, biasr, m_in, m_out, append_ones):
    grid = (pl.cdiv(N, TN),)
    w_out = PW if append_ones else m_out
    return pl.pallas_call(
        functools.partial(_upd_body, append_ones=append_ones),
        grid=grid,
        in_specs=[
            pl.BlockSpec((TN, m_out), lambda i: (i, 0)),
            pl.BlockSpec((TN, m_out), lambda i: (i, 0)),
            pl.BlockSpec((TN, m_in), lambda i: (i, 0)),
            pl.BlockSpec((m_in, m_out), lambda i: (0, 0)),
            pl.BlockSpec((1, m_out), lambda i: (0, 0)),
        ],
        out_specs=pl.BlockSpec((TN, w_out), lambda i: (i, 0)),
        out_shape=jax.ShapeDtypeStruct((N, w_out), jnp.float32),
    )(p0, p1, x, root, biasr)


def _final_body(p0_ref, p1_ref, s2g_ref, fc1w_ref, fc1b_ref, fc2w_ref,
                fc2b_ref, fc3w_ref, fc3b_ref, out_ref):
    tot = p0_ref[...] + p1_ref[...]
    cnt = jnp.maximum(tot[:, 64:65], 1.0)
    mean1 = tot[:, :64] / cnt
    gids = s2g_ref[...]
    onehot = jnp.where(
        lax.broadcasted_iota(jnp.int32, (G, S), 0) == gids, 1.0, 0.0)
    sums2 = jnp.dot(onehot, mean1, preferred_element_type=jnp.float32)
    cnt2 = jnp.maximum(jnp.sum(onehot, axis=1, keepdims=True), 1.0)
    mean2 = sums2 / cnt2
    a = _elu(jnp.dot(mean2, fc1w_ref[...],
                     preferred_element_type=jnp.float32) + fc1b_ref[...])
    b = _elu(jnp.dot(a, fc2w_ref[...],
                     preferred_element_type=jnp.float32) + fc2b_ref[...])
    out_ref[...] = (jnp.dot(b, fc3w_ref[...],
                            preferred_element_type=jnp.float32)
                    + fc3b_ref[...])


def _final_call(p0, p1, s2g2d, fc1w, fc1b, fc2w, fc2b, fc3w, fc3b):
    return pl.pallas_call(
        _final_body,
        out_shape=jax.ShapeDtypeStruct((G, 1), jnp.float32),
    )(p0, p1, s2g2d, fc1w, fc1b, fc2w, fc2b, fc3w, fc3b)


_gather = {16: _make_gather(16), 32: _make_gather(32), 64: _make_gather(64)}
_scatter = {32: _make_scatter(32), 64: _make_scatter(64)}


def kernel(x, edge_index, edge_attr, node_to_subgraph, subgraph_to_graph,
           nn1_W1, nn1_b1, nn1_W2, nn1_b2, root1, bias1,
           nn2_W1, nn2_b1, nn2_W2, nn2_b2, root2, bias2,
           nn3_W1, nn3_b1, nn3_W2, nn3_b2, root3, bias3,
           fc1_W, fc1_b, fc2_W, fc2_b, fc3_W, fc3_b):
    f32 = jnp.float32
    i32 = jnp.int32
    pad = EP - E
    src3 = jnp.concatenate(
        [edge_index[0], jnp.zeros((pad,), edge_index.dtype)]
    ).astype(i32).reshape(NW, NCH, CH)
    dst3 = jnp.concatenate(
        [edge_index[1], jnp.zeros((pad,), edge_index.dtype)]
    ).astype(i32).reshape(NW, NCH, CH)
    ea_p = jnp.concatenate(
        [edge_attr.astype(f32), jnp.zeros((pad, EA), f32)])
    n2s3 = node_to_subgraph.astype(i32).reshape(NW, PNCH, PCH)
    s2g2d = subgraph_to_graph.astype(i32).reshape(1, S)

    zN64 = jnp.zeros((N, 64), f32)
    zS = jnp.zeros((S, PW), f32)

    layers = (
        (D, 32, nn1_W1, nn1_b1, nn1_W2, nn1_b2, root1, bias1),
        (32, 64, nn2_W1, nn2_b1, nn2_W2, nn2_b2, root2, bias2),
        (64, 64, nn3_W1, nn3_b1, nn3_W2, nn3_b2, root3, bias3),
    )
    h = x.astype(f32)
    for li, (m_in, m_out, w1, b1, w2, b2, root, bias) in enumerate(layers):
        xs = _gather[m_in](h, src3)
        msg = _msg_call(ea_p, xs, w1.astype(f32), b1.reshape(1, H),
                        w2.astype(jnp.bfloat16), b2.reshape(m_in, m_out),
                        m_in, m_out)
        p = _scatter[m_out](zN64[:, :m_out], msg, dst3)
        h = _upd_call(p[:N], p[N:], h, root.astype(f32),
                      bias.reshape(1, m_out), m_in, m_out,
                      append_ones=(li == 2))

    pp = _pool_k(zS, h, n2s3)
    out = _final_call(pp[:S], pp[S:], s2g2d,
                      fc1_W.astype(f32), fc1_b.reshape(1, 32),
                      fc2_W.astype(f32), fc2_b.reshape(1, 16),
                      fc3_W.astype(f32), fc3_b.reshape(1, 1))
    return out.reshape(-1)


# speedup vs baseline: 1.0320x; 1.0320x over previous
import functools

import jax
import jax.numpy as jnp
from jax import lax
from jax.experimental import pallas as pl
from jax.experimental.pallas import tpu as pltpu
from jax.experimental.pallas import tpu_sc as plsc

N = 20000
E = 60000
D = 16
S = 2000
G = 64
H = 128
EA = 5

NC = 2
NS = 16
NW = NC * NS
CH = 118
NCH = 16
EPW = NCH * CH
EP = NW * EPW

NPW = N // NW
PCH = 125
PNCH = 5
NPT = N // NS
SPT = S // NS
PW = 80

TE = 256
TN = 256


def _mesh():
    return plsc.VectorSubcoreMesh(
        core_axis_name="c", subcore_axis_name="s",
        num_cores=NC, num_subcores=NS)


@functools.lru_cache(maxsize=None)
def _make_gather(d):
    @functools.partial(
        pl.kernel,
        mesh=_mesh(),
        compiler_params=pltpu.CompilerParams(use_tc_tiling_on_sc=False),
        out_type=jax.ShapeDtypeStruct((EP, d), jnp.float32),
        scratch_types=[
            pltpu.VMEM((NCH, CH), jnp.int32),
            pltpu.VMEM((EPW, d), jnp.float32),
            pltpu.SemaphoreType.DMA,
        ],
    )
    def gather_k(table, idx, out, idx_v, rows_v, sem):
        wid = lax.axis_index("c") * NS + lax.axis_index("s")
        pltpu.sync_copy(idx.at[wid], idx_v)
        copies = [
            pltpu.async_copy(
                table.at[idx_v.at[j]], rows_v.at[pl.ds(j * CH, CH)], sem
            )
            for j in range(NCH)
        ]
        for cp in copies:
            cp.wait()
        pltpu.sync_copy(rows_v, out.at[pl.ds(wid * EPW, EPW)])

    return gather_k


@functools.lru_cache(maxsize=None)
def _make_scatter(d):
    @functools.partial(
        pl.kernel,
        mesh=_mesh(),
        compiler_params=pltpu.CompilerParams(use_tc_tiling_on_sc=False),
        out_type=jax.ShapeDtypeStruct((NC * N, d), jnp.float32),
        scratch_types=[
            pltpu.VMEM((NCH, CH), jnp.int32),
            pltpu.VMEM((2, CH, d), jnp.float32),
            pltpu.VMEM_SHARED((N, d), jnp.float32),
            pltpu.SemaphoreType.DMA,
        ],
    )
    def scatter_k(zeros, msg, idx, out, idx_v, buf, acc, sem):
        c = lax.axis_index("c")
        s = lax.axis_index("s")
        wid = c * NS + s
        pltpu.sync_copy(zeros.at[pl.ds(s * NPT, NPT)], acc.at[pl.ds(s * NPT, NPT)])
        pltpu.sync_copy(idx.at[wid], idx_v)
        plsc.subcore_barrier()
        prev = pltpu.async_copy(
            msg.at[pl.ds(wid * EPW, CH)], buf.at[0], sem)
        for j in range(NCH):
            if j + 1 < NCH:
                nxt = pltpu.async_copy(
                    msg.at[pl.ds(wid * EPW + (j + 1) * CH, CH)],
                    buf.at[(j + 1) % 2], sem)
            prev.wait()
            pltpu.sync_copy(buf.at[j % 2], acc.at[idx_v.at[j]], add=True)
            if j + 1 < NCH:
                prev = nxt
        plsc.subcore_barrier()
        pltpu.sync_copy(
            acc.at[pl.ds(s * NPT, NPT)], out.at[pl.ds(c * N + s * NPT, NPT)]
        )

    return scatter_k


@functools.lru_cache(maxsize=None)
def _make_pool():
    @functools.partial(
        pl.kernel,
        mesh=_mesh(),
        compiler_params=pltpu.CompilerParams(use_tc_tiling_on_sc=False),
        out_type=jax.ShapeDtypeStruct((NC * S, PW), jnp.float32),
        scratch_types=[
            pltpu.VMEM((PNCH, PCH), jnp.int32),
            pltpu.VMEM((NPW, PW), jnp.float32),
            pltpu.VMEM_SHARED((S, PW), jnp.float32),
            pltpu.SemaphoreType.DMA,
        ],
    )
    def pool_k(zeros, rows, idx, out, idx_v, rows_v, acc, sem):
        c = lax.axis_index("c")
        s = lax.axis_index("s")
        wid = c * NS + s
        pltpu.sync_copy(zeros.at[pl.ds(s * SPT, SPT)],
                        acc.at[pl.ds(s * SPT, SPT)])
        pltpu.sync_copy(idx.at[wid], idx_v)
        pltpu.sync_copy(rows.at[pl.ds(wid * NPW, NPW)], rows_v)
        plsc.subcore_barrier()
        for j in range(PNCH):
            pltpu.sync_copy(
                rows_v.at[pl.ds(j * PCH, PCH)], acc.at[idx_v.at[j]], add=True
            )
        plsc.subcore_barrier()
        pltpu.sync_copy(
            acc.at[pl.ds(s * SPT, SPT)], out.at[pl.ds(c * S + s * SPT, SPT)]
        )

    return pool_k


def _msg_body(ea_ref, xs_ref, w1_ref, b1_ref, w2_ref, b2r_ref, out_ref,
              *, m_in, m_out):
    i = pl.program_id(0)
    ea = ea_ref[...]
    h = jnp.maximum(
        jnp.dot(ea, w1_ref[...], preferred_element_type=jnp.float32)
        + b1_ref[...],
        0.0,
    )
    wt = jnp.dot(h.astype(jnp.bfloat16), w2_ref[...],
                 preferred_element_type=jnp.float32)
    xs = xs_ref[...]
    acc = jnp.dot(xs, b2r_ref[...], preferred_element_type=jnp.float32)
    for k in range(m_in):
        acc = acc + xs[:, k:k + 1] * wt[:, k * m_out:(k + 1) * m_out]
    row = i * TE + lax.broadcasted_iota(jnp.int32, (TE, 1), 0)
    out_ref[...] = jnp.where(row < E, acc, 0.0)


def _msg_call(ea_p, xs, w1, b1r, w2b, b2r, m_in, m_out):
    grid = (EP // TE,)
    return pl.pallas_call(
        functools.partial(_msg_body, m_in=m_in, m_out=m_out),
        grid=grid,
        in_specs=[
            pl.BlockSpec((TE, EA), lambda i: (i, 0)),
            pl.BlockSpec((TE, m_in), lambda i: (i, 0)),
            pl.BlockSpec((EA, H), lambda i: (0, 0)),
            pl.BlockSpec((1, H), lambda i: (0, 0)),
            pl.BlockSpec((H, m_in * m_out), lambda i: (0, 0)),
            pl.BlockSpec((m_in, m_out), lambda i: (0, 0)),
        ],
        out_specs=pl.BlockSpec((TE, m_out), lambda i: (i, 0)),
        out_shape=jax.ShapeDtypeStruct((EP, m_out), jnp.float32),
    )(ea_p, xs, w1, b1r, w2b, b2r)


def _elu(y):
    return jnp.where(y > 0, y, jnp.exp(jnp.minimum(y, 0.0)) - 1.0)


def _upd_body(p0_ref, p1_ref, x_ref, root_ref, bias_ref, out_ref,
              *, append_ones):
    y = (p0_ref[...] + p1_ref[...]
         + jnp.dot(x_ref[...], root_ref[...],
                   preferred_element_type=jnp.float32)
         + bias_ref[...])
    hout = _elu(y)
    if append_ones:
        out_ref[...] = jnp.concatenate(
            [hout, jnp.ones((hout.shape[0], PW - 64), jnp.float32)], axis=1)
    else:
        out_ref[...] = hout


def _upd_call(p0, p1, x, root, biasr, m_in, m_out, append_ones):
    grid = (pl.cdiv(N, TN),)
    w_out = PW if append_ones else m_out
    return pl.pallas_call(
        functools.partial(_upd_body, append_ones=append_ones),
        grid=grid,
        in_specs=[
            pl.BlockSpec((TN, m_out), lambda i: (i, 0)),
            pl.BlockSpec((TN, m_out), lambda i: (i, 0)),
            pl.BlockSpec((TN, m_in), lambda i: (i, 0)),
            pl.BlockSpec((m_in, m_out), lambda i: (0, 0)),
            pl.BlockSpec((1, m_out), lambda i: (0, 0)),
        ],
        out_specs=pl.BlockSpec((TN, w_out), lambda i: (i, 0)),
        out_shape=jax.ShapeDtypeStruct((N, w_out), jnp.float32),
    )(p0, p1, x, root, biasr)


def _final_body(p0_ref, p1_ref, s2g_ref, fc1w_ref, fc1b_ref, fc2w_ref,
                fc2b_ref, fc3w_ref, fc3b_ref, out_ref):
    tot = p0_ref[...] + p1_ref[...]
    cnt = jnp.maximum(tot[:, 64:65], 1.0)
    mean1 = tot[:, :64] / cnt
    gids = s2g_ref[...]
    onehot = jnp.where(
        lax.broadcasted_iota(jnp.int32, (G, S), 0) == gids, 1.0, 0.0)
    sums2 = jnp.dot(onehot, mean1, preferred_element_type=jnp.float32)
    cnt2 = jnp.maximum(jnp.sum(onehot, axis=1, keepdims=True), 1.0)
    mean2 = sums2 / cnt2
    a = _elu(jnp.dot(mean2, fc1w_ref[...],
                     preferred_element_type=jnp.float32) + fc1b_ref[...])
    b = _elu(jnp.dot(a, fc2w_ref[...],
                     preferred_element_type=jnp.float32) + fc2b_ref[...])
    out_ref[...] = (jnp.dot(b, fc3w_ref[...],
                            preferred_element_type=jnp.float32)
                    + fc3b_ref[...])


def _final_call(p0, p1, s2g2d, fc1w, fc1b, fc2w, fc2b, fc3w, fc3b):
    return pl.pallas_call(
        _final_body,
        out_shape=jax.ShapeDtypeStruct((G, 1), jnp.float32),
    )(p0, p1, s2g2d, fc1w, fc1b, fc2w, fc2b, fc3w, fc3b)


def kernel(x, edge_index, edge_attr, node_to_subgraph, subgraph_to_graph,
           nn1_W1, nn1_b1, nn1_W2, nn1_b2, root1, bias1,
           nn2_W1, nn2_b1, nn2_W2, nn2_b2, root2, bias2,
           nn3_W1, nn3_b1, nn3_W2, nn3_b2, root3, bias3,
           fc1_W, fc1_b, fc2_W, fc2_b, fc3_W, fc3_b):
    f32 = jnp.float32
    i32 = jnp.int32
    pad = EP - E
    src3 = jnp.concatenate(
        [edge_index[0], jnp.zeros((pad,), edge_index.dtype)]
    ).astype(i32).reshape(NW, NCH, CH)
    dst3 = jnp.concatenate(
        [edge_index[1], jnp.zeros((pad,), edge_index.dtype)]
    ).astype(i32).reshape(NW, NCH, CH)
    ea_p = jnp.concatenate(
        [edge_attr.astype(f32), jnp.zeros((pad, EA), f32)])
    n2s3 = node_to_subgraph.astype(i32).reshape(NW, PNCH, PCH)
    s2g2d = subgraph_to_graph.astype(i32).reshape(1, S)

    zN64 = jnp.zeros((N, 64), f32)
    zS = jnp.zeros((S, PW), f32)

    layers = (
        (D, 32, nn1_W1, nn1_b1, nn1_W2, nn1_b2, root1, bias1),
        (32, 64, nn2_W1, nn2_b1, nn2_W2, nn2_b2, root2, bias2),
        (64, 64, nn3_W1, nn3_b1, nn3_W2, nn3_b2, root3, bias3),
    )
    h = x.astype(f32)
    for li, (m_in, m_out, w1, b1, w2, b2, root, bias) in enumerate(layers):
        xs = _make_gather(m_in)(h, src3)
        msg = _msg_call(ea_p, xs, w1.astype(f32), b1.reshape(1, H),
                        w2.astype(jnp.bfloat16), b2.reshape(m_in, m_out),
                        m_in, m_out)
        p = _make_scatter(m_out)(zN64[:, :m_out], msg, dst3)
        h = _upd_call(p[:N], p[N:], h, root.astype(f32),
                      bias.reshape(1, m_out), m_in, m_out,
                      append_ones=(li == 2))

    pp = _make_pool()(zS, h, n2s3)
    out = _final_call(pp[:S], pp[S:], s2g2d,
                      fc1_W.astype(f32), fc1_b.reshape(1, 32),
                      fc2_W.astype(f32), fc2_b.reshape(1, 16),
                      fc3_W.astype(f32), fc3_b.reshape(1, 1))
    return out.reshape(-1)


# speedup vs baseline: 1.7105x; 1.6575x over previous
import functools

import jax
import jax.numpy as jnp
from jax import lax
from jax.experimental import pallas as pl
from jax.experimental.pallas import tpu as pltpu
from jax.experimental.pallas import tpu_sc as plsc

N = 20000
E = 60000
D = 16
S = 2000
G = 64
H = 128
EA = 5

NC = 2
NS = 16
NW = NC * NS
CH = 118
NCH = 16
EPW = NCH * CH
EP = NW * EPW

NPW = N // NW
PCH = 125
PNCH = 5
NPT = N // NS
SPT = S // NS
PW = 80

TE = 256
TN = 256


def _mesh():
    return plsc.VectorSubcoreMesh(
        core_axis_name="c", subcore_axis_name="s",
        num_cores=NC, num_subcores=NS)


@functools.lru_cache(maxsize=None)
def _make_gather(d):
    @functools.partial(
        pl.kernel,
        mesh=_mesh(),
        compiler_params=pltpu.CompilerParams(use_tc_tiling_on_sc=False),
        out_type=jax.ShapeDtypeStruct((EP, d), jnp.float32),
        scratch_types=[
            pltpu.VMEM((NCH, CH), jnp.int32),
            pltpu.VMEM((EPW, d), jnp.float32),
            pltpu.SemaphoreType.DMA,
        ],
    )
    def gather_k(table, idx, out, idx_v, rows_v, sem):
        wid = lax.axis_index("c") * NS + lax.axis_index("s")
        pltpu.sync_copy(idx.at[wid], idx_v)
        copies = [
            pltpu.async_copy(
                table.at[idx_v.at[j]], rows_v.at[pl.ds(j * CH, CH)], sem
            )
            for j in range(NCH)
        ]
        for cp in copies:
            cp.wait()
        pltpu.sync_copy(rows_v, out.at[pl.ds(wid * EPW, EPW)])

    return gather_k


@functools.lru_cache(maxsize=None)
def _make_scatter(d):
    @functools.partial(
        pl.kernel,
        mesh=_mesh(),
        compiler_params=pltpu.CompilerParams(use_tc_tiling_on_sc=False),
        out_type=jax.ShapeDtypeStruct((NC * N, d), jnp.float32),
        scratch_types=[
            pltpu.VMEM((NCH, CH), jnp.int32),
            pltpu.VMEM((2, CH, d), jnp.float32),
            pltpu.VMEM_SHARED((N, d), jnp.float32),
            pltpu.SemaphoreType.DMA,
        ],
    )
    def scatter_k(zeros, msg, idx, out, idx_v, buf, acc, sem):
        c = lax.axis_index("c")
        s = lax.axis_index("s")
        wid = c * NS + s
        pltpu.sync_copy(zeros.at[pl.ds(s * NPT, NPT)], acc.at[pl.ds(s * NPT, NPT)])
        pltpu.sync_copy(idx.at[wid], idx_v)
        plsc.subcore_barrier()
        prev = pltpu.async_copy(
            msg.at[pl.ds(wid * EPW, CH)], buf.at[0], sem)
        for j in range(NCH):
            if j + 1 < NCH:
                nxt = pltpu.async_copy(
                    msg.at[pl.ds(wid * EPW + (j + 1) * CH, CH)],
                    buf.at[(j + 1) % 2], sem)
            prev.wait()
            pltpu.sync_copy(buf.at[j % 2], acc.at[idx_v.at[j]], add=True)
            if j + 1 < NCH:
                prev = nxt
        plsc.subcore_barrier()
        pltpu.sync_copy(
            acc.at[pl.ds(s * NPT, NPT)], out.at[pl.ds(c * N + s * NPT, NPT)]
        )

    return scatter_k


@functools.lru_cache(maxsize=None)
def _make_pool():
    @functools.partial(
        pl.kernel,
        mesh=_mesh(),
        compiler_params=pltpu.CompilerParams(use_tc_tiling_on_sc=False),
        out_type=jax.ShapeDtypeStruct((NC * S, PW), jnp.float32),
        scratch_types=[
            pltpu.VMEM((PNCH, PCH), jnp.int32),
            pltpu.VMEM((NPW, PW), jnp.float32),
            pltpu.VMEM_SHARED((S, PW), jnp.float32),
            pltpu.SemaphoreType.DMA,
        ],
    )
    def pool_k(zeros, rows, idx, out, idx_v, rows_v, acc, sem):
        c = lax.axis_index("c")
        s = lax.axis_index("s")
        wid = c * NS + s
        pltpu.sync_copy(zeros.at[pl.ds(s * SPT, SPT)],
                        acc.at[pl.ds(s * SPT, SPT)])
        pltpu.sync_copy(idx.at[wid], idx_v)
        pltpu.sync_copy(rows.at[pl.ds(wid * NPW, NPW)], rows_v)
        plsc.subcore_barrier()
        for j in range(PNCH):
            pltpu.sync_copy(
                rows_v.at[pl.ds(j * PCH, PCH)], acc.at[idx_v.at[j]], add=True
            )
        plsc.subcore_barrier()
        pltpu.sync_copy(
            acc.at[pl.ds(s * SPT, SPT)], out.at[pl.ds(c * S + s * SPT, SPT)]
        )

    return pool_k


def _msg_body(ea_ref, xs_ref, w1_ref, b1_ref, w2_ref, b2r_ref, r_ref, rt_ref,
              out_ref, *, m_in, m_out):
    i = pl.program_id(0)
    ea = ea_ref[...]
    h = jnp.maximum(
        jnp.dot(ea, w1_ref[...], preferred_element_type=jnp.float32)
        + b1_ref[...],
        0.0,
    )
    wt = jnp.dot(h.astype(jnp.bfloat16), w2_ref[...],
                 preferred_element_type=jnp.float32)
    xs = xs_ref[...]
    xb = jnp.dot(xs.astype(jnp.bfloat16), r_ref[...],
                 preferred_element_type=jnp.float32)
    prod = (xb * wt).astype(jnp.bfloat16)
    acc = jnp.dot(prod, rt_ref[...], preferred_element_type=jnp.float32)
    acc = acc + jnp.dot(xs, b2r_ref[...], preferred_element_type=jnp.float32)
    row = i * TE + lax.broadcasted_iota(jnp.int32, (TE, 1), 0)
    out_ref[...] = jnp.where(row < E, acc, 0.0)


def _msg_call(ea_p, xs, w1, b1r, w2b, b2r, m_in, m_out):
    grid = (EP // TE,)
    kk = jnp.arange(m_in * m_out, dtype=jnp.int32) // m_out
    oo = jnp.arange(m_in * m_out, dtype=jnp.int32) % m_out
    r = (jnp.arange(m_in, dtype=jnp.int32)[:, None] == kk[None, :]
         ).astype(jnp.bfloat16)
    rt = (oo[:, None] == jnp.arange(m_out, dtype=jnp.int32)[None, :]
          ).astype(jnp.bfloat16)
    return pl.pallas_call(
        functools.partial(_msg_body, m_in=m_in, m_out=m_out),
        grid=grid,
        in_specs=[
            pl.BlockSpec((TE, EA), lambda i: (i, 0)),
            pl.BlockSpec((TE, m_in), lambda i: (i, 0)),
            pl.BlockSpec((EA, H), lambda i: (0, 0)),
            pl.BlockSpec((1, H), lambda i: (0, 0)),
            pl.BlockSpec((H, m_in * m_out), lambda i: (0, 0)),
            pl.BlockSpec((m_in, m_out), lambda i: (0, 0)),
            pl.BlockSpec((m_in, m_in * m_out), lambda i: (0, 0)),
            pl.BlockSpec((m_in * m_out, m_out), lambda i: (0, 0)),
        ],
        out_specs=pl.BlockSpec((TE, m_out), lambda i: (i, 0)),
        out_shape=jax.ShapeDtypeStruct((EP, m_out), jnp.float32),
    )(ea_p, xs, w1, b1r, w2b, b2r, r, rt)


def _elu(y):
    return jnp.where(y > 0, y, jnp.exp(jnp.minimum(y, 0.0)) - 1.0)


def _upd_body(p0_ref, p1_ref, x_ref, root_ref, bias_ref, out_ref,
              *, append_ones):
    y = (p0_ref[...] + p1_ref[...]
         + jnp.dot(x_ref[...], root_ref[...],
                   preferred_element_type=jnp.float32)
         + bias_ref[...])
    hout = _elu(y)
    if append_ones:
        out_ref[...] = jnp.concatenate(
            [hout, jnp.ones((hout.shape[0], PW - 64), jnp.float32)], axis=1)
    else:
        out_ref[...] = hout


def _upd_call(p0, p1, x, root, biasr, m_in, m_out, append_ones):
    grid = (pl.cdiv(N, TN),)
    w_out = PW if append_ones else m_out
    return pl.pallas_call(
        functools.partial(_upd_body, append_ones=append_ones),
        grid=grid,
        in_specs=[
            pl.BlockSpec((TN, m_out), lambda i: (i, 0)),
            pl.BlockSpec((TN, m_out), lambda i: (i, 0)),
            pl.BlockSpec((TN, m_in), lambda i: (i, 0)),
            pl.BlockSpec((m_in, m_out), lambda i: (0, 0)),
            pl.BlockSpec((1, m_out), lambda i: (0, 0)),
        ],
        out_specs=pl.BlockSpec((TN, w_out), lambda i: (i, 0)),
        out_shape=jax.ShapeDtypeStruct((N, w_out), jnp.float32),
    )(p0, p1, x, root, biasr)


def _final_body(p0_ref, p1_ref, s2g_ref, fc1w_ref, fc1b_ref, fc2w_ref,
                fc2b_ref, fc3w_ref, fc3b_ref, out_ref):
    tot = p0_ref[...] + p1_ref[...]
    cnt = jnp.maximum(tot[:, 64:65], 1.0)
    mean1 = tot[:, :64] / cnt
    gids = s2g_ref[...]
    onehot = jnp.where(
        lax.broadcasted_iota(jnp.int32, (G, S), 0) == gids, 1.0, 0.0)
    sums2 = jnp.dot(onehot, mean1, preferred_element_type=jnp.float32)
    cnt2 = jnp.maximum(jnp.sum(onehot, axis=1, keepdims=True), 1.0)
    mean2 = sums2 / cnt2
    a = _elu(jnp.dot(mean2, fc1w_ref[...],
                     preferred_element_type=jnp.float32) + fc1b_ref[...])
    b = _elu(jnp.dot(a, fc2w_ref[...],
                     preferred_element_type=jnp.float32) + fc2b_ref[...])
    out_ref[...] = (jnp.dot(b, fc3w_ref[...],
                            preferred_element_type=jnp.float32)
                    + fc3b_ref[...])


def _final_call(p0, p1, s2g2d, fc1w, fc1b, fc2w, fc2b, fc3w, fc3b):
    return pl.pallas_call(
        _final_body,
        out_shape=jax.ShapeDtypeStruct((G, 1), jnp.float32),
    )(p0, p1, s2g2d, fc1w, fc1b, fc2w, fc2b, fc3w, fc3b)


def kernel(x, edge_index, edge_attr, node_to_subgraph, subgraph_to_graph,
           nn1_W1, nn1_b1, nn1_W2, nn1_b2, root1, bias1,
           nn2_W1, nn2_b1, nn2_W2, nn2_b2, root2, bias2,
           nn3_W1, nn3_b1, nn3_W2, nn3_b2, root3, bias3,
           fc1_W, fc1_b, fc2_W, fc2_b, fc3_W, fc3_b):
    f32 = jnp.float32
    i32 = jnp.int32
    pad = EP - E
    src3 = jnp.concatenate(
        [edge_index[0], jnp.zeros((pad,), edge_index.dtype)]
    ).astype(i32).reshape(NW, NCH, CH)
    dst3 = jnp.concatenate(
        [edge_index[1], jnp.zeros((pad,), edge_index.dtype)]
    ).astype(i32).reshape(NW, NCH, CH)
    ea_p = jnp.concatenate(
        [edge_attr.astype(f32), jnp.zeros((pad, EA), f32)])
    n2s3 = node_to_subgraph.astype(i32).reshape(NW, PNCH, PCH)
    s2g2d = subgraph_to_graph.astype(i32).reshape(1, S)

    zN64 = jnp.zeros((N, 64), f32)
    zS = jnp.zeros((S, PW), f32)

    layers = (
        (D, 32, nn1_W1, nn1_b1, nn1_W2, nn1_b2, root1, bias1),
        (32, 64, nn2_W1, nn2_b1, nn2_W2, nn2_b2, root2, bias2),
        (64, 64, nn3_W1, nn3_b1, nn3_W2, nn3_b2, root3, bias3),
    )
    h = x.astype(f32)
    for li, (m_in, m_out, w1, b1, w2, b2, root, bias) in enumerate(layers):
        xs = _make_gather(m_in)(h, src3)
        msg = _msg_call(ea_p, xs, w1.astype(f32), b1.reshape(1, H),
                        w2.astype(jnp.bfloat16), b2.reshape(m_in, m_out),
                        m_in, m_out)
        p = _make_scatter(m_out)(zN64[:, :m_out], msg, dst3)
        h = _upd_call(p[:N], p[N:], h, root.astype(f32),
                      bias.reshape(1, m_out), m_in, m_out,
                      append_ones=(li == 2))

    pp = _make_pool()(zS, h, n2s3)
    out = _final_call(pp[:S], pp[S:], s2g2d,
                      fc1_W.astype(f32), fc1_b.reshape(1, 32),
                      fc2_W.astype(f32), fc2_b.reshape(1, 16),
                      fc3_W.astype(f32), fc3_b.reshape(1, 1))
    return out.reshape(-1)


# speedup vs baseline: 1.8068x; 1.0563x over previous
import functools

import jax
import jax.numpy as jnp
from jax import lax
from jax.experimental import pallas as pl
from jax.experimental.pallas import tpu as pltpu
from jax.experimental.pallas import tpu_sc as plsc

N = 20000
E = 60000
D = 16
S = 2000
G = 64
H = 128
EA = 5

NC = 2
NS = 16
NW = NC * NS
CH = 118
NCH = 16
EPW = NCH * CH
EP = NW * EPW

NPW = N // NW
PCH = 125
PNCH = 5
NPT = N // NS
SPT = S // NS
PW = 80

TE = 256
TN = 256


def _mesh():
    return plsc.VectorSubcoreMesh(
        core_axis_name="c", subcore_axis_name="s",
        num_cores=NC, num_subcores=NS)


@functools.lru_cache(maxsize=None)
def _make_gather(d):
    @functools.partial(
        pl.kernel,
        mesh=_mesh(),
        compiler_params=pltpu.CompilerParams(use_tc_tiling_on_sc=False),
        out_type=jax.ShapeDtypeStruct((EP, d), jnp.float32),
        scratch_types=[
            pltpu.VMEM((NCH, CH), jnp.int32),
            pltpu.VMEM((EPW, d), jnp.float32),
            pltpu.SemaphoreType.DMA,
        ],
    )
    def gather_k(table, idx, out, idx_v, rows_v, sem):
        wid = lax.axis_index("c") * NS + lax.axis_index("s")
        pltpu.sync_copy(idx.at[wid], idx_v)
        copies = [
            pltpu.async_copy(
                table.at[idx_v.at[j]], rows_v.at[pl.ds(j * CH, CH)], sem
            )
            for j in range(NCH)
        ]
        for cp in copies:
            cp.wait()
        pltpu.sync_copy(rows_v, out.at[pl.ds(wid * EPW, EPW)])

    return gather_k


@functools.lru_cache(maxsize=None)
def _make_scatter(d):
    @functools.partial(
        pl.kernel,
        mesh=_mesh(),
        compiler_params=pltpu.CompilerParams(use_tc_tiling_on_sc=False),
        out_type=jax.ShapeDtypeStruct((NC * N, d), jnp.float32),
        scratch_types=[
            pltpu.VMEM((NCH, CH), jnp.int32),
            pltpu.VMEM((2, CH, d), jnp.float32),
            pltpu.VMEM_SHARED((N, d), jnp.float32),
            pltpu.SemaphoreType.DMA,
        ],
    )
    def scatter_k(zeros, msg, idx, out, idx_v, buf, acc, sem):
        c = lax.axis_index("c")
        s = lax.axis_index("s")
        wid = c * NS + s
        pltpu.sync_copy(zeros.at[pl.ds(s * NPT, NPT)], acc.at[pl.ds(s * NPT, NPT)])
        pltpu.sync_copy(idx.at[wid], idx_v)
        plsc.subcore_barrier()
        prev = pltpu.async_copy(
            msg.at[pl.ds(wid * EPW, CH)], buf.at[0], sem)
        for j in range(NCH):
            if j + 1 < NCH:
                nxt = pltpu.async_copy(
                    msg.at[pl.ds(wid * EPW + (j + 1) * CH, CH)],
                    buf.at[(j + 1) % 2], sem)
            prev.wait()
            pltpu.sync_copy(buf.at[j % 2], acc.at[idx_v.at[j]], add=True)
            if j + 1 < NCH:
                prev = nxt
        plsc.subcore_barrier()
        pltpu.sync_copy(
            acc.at[pl.ds(s * NPT, NPT)], out.at[pl.ds(c * N + s * NPT, NPT)]
        )

    return scatter_k


@functools.lru_cache(maxsize=None)
def _make_pool():
    @functools.partial(
        pl.kernel,
        mesh=_mesh(),
        compiler_params=pltpu.CompilerParams(use_tc_tiling_on_sc=False),
        out_type=jax.ShapeDtypeStruct((NC * S, PW), jnp.float32),
        scratch_types=[
            pltpu.VMEM((PNCH, PCH), jnp.int32),
            pltpu.VMEM((NPW, PW), jnp.float32),
            pltpu.VMEM_SHARED((S, PW), jnp.float32),
            pltpu.SemaphoreType.DMA,
        ],
    )
    def pool_k(zeros, rows, idx, out, idx_v, rows_v, acc, sem):
        c = lax.axis_index("c")
        s = lax.axis_index("s")
        wid = c * NS + s
        pltpu.sync_copy(zeros.at[pl.ds(s * SPT, SPT)],
                        acc.at[pl.ds(s * SPT, SPT)])
        pltpu.sync_copy(idx.at[wid], idx_v)
        pltpu.sync_copy(rows.at[pl.ds(wid * NPW, NPW)], rows_v)
        plsc.subcore_barrier()
        for j in range(PNCH):
            pltpu.sync_copy(
                rows_v.at[pl.ds(j * PCH, PCH)], acc.at[idx_v.at[j]], add=True
            )
        plsc.subcore_barrier()
        pltpu.sync_copy(
            acc.at[pl.ds(s * SPT, SPT)], out.at[pl.ds(c * S + s * SPT, SPT)]
        )

    return pool_k


def _msg_body(ea_ref, xs_ref, w1_ref, b1_ref, w2_ref, b2r_ref, r_ref, rt_ref,
              out_ref, *, m_in, m_out):
    i = pl.program_id(0)
    fi = 128 // m_in
    fo = 128 // m_out
    ea = ea_ref[...]
    h = jnp.maximum(
        jnp.dot(ea, w1_ref[...], preferred_element_type=jnp.float32)
        + b1_ref[...],
        0.0,
    )
    wt = jnp.dot(h.astype(jnp.bfloat16), w2_ref[...],
                 preferred_element_type=jnp.float32)
    x128 = xs_ref[...]
    xs = jnp.concatenate(
        [x128[:, j * m_in:(j + 1) * m_in] for j in range(fi)], axis=0)
    xb = jnp.dot(xs.astype(jnp.bfloat16), r_ref[...],
                 preferred_element_type=jnp.float32)
    prod = (xb * wt).astype(jnp.bfloat16)
    acc = jnp.dot(prod, rt_ref[...], preferred_element_type=jnp.float32)
    acc = acc + jnp.dot(xs, b2r_ref[...], preferred_element_type=jnp.float32)
    row = i * TE + lax.broadcasted_iota(jnp.int32, (TE, 1), 0)
    acc = jnp.where(row < E, acc, 0.0)
    out_ref[...] = jnp.concatenate(
        [acc[j * (TE // fo):(j + 1) * (TE // fo), :] for j in range(fo)],
        axis=1)


def _msg_call(ea_p, xs_pk, w1, b1r, w2b, b2r, m_in, m_out):
    grid = (EP // TE,)
    kk = jnp.arange(m_in * m_out, dtype=jnp.int32) // m_out
    oo = jnp.arange(m_in * m_out, dtype=jnp.int32) % m_out
    r = (jnp.arange(m_in, dtype=jnp.int32)[:, None] == kk[None, :]
         ).astype(jnp.bfloat16)
    rt = (oo[:, None] == jnp.arange(m_out, dtype=jnp.int32)[None, :]
          ).astype(jnp.bfloat16)
    fi = 128 // m_in
    fo = 128 // m_out
    return pl.pallas_call(
        functools.partial(_msg_body, m_in=m_in, m_out=m_out),
        grid=grid,
        in_specs=[
            pl.BlockSpec((TE, EA), lambda i: (i, 0)),
            pl.BlockSpec((TE // fi, 128), lambda i: (i, 0)),
            pl.BlockSpec((EA, H), lambda i: (0, 0)),
            pl.BlockSpec((1, H), lambda i: (0, 0)),
            pl.BlockSpec((H, m_in * m_out), lambda i: (0, 0)),
            pl.BlockSpec((m_in, m_out), lambda i: (0, 0)),
            pl.BlockSpec((m_in, m_in * m_out), lambda i: (0, 0)),
            pl.BlockSpec((m_in * m_out, m_out), lambda i: (0, 0)),
        ],
        out_specs=pl.BlockSpec((TE // fo, 128), lambda i: (i, 0)),
        out_shape=jax.ShapeDtypeStruct((EP // fo, 128), jnp.float32),
    )(ea_p, xs_pk, w1, b1r, w2b, b2r, r, rt)


def _elu(y):
    return jnp.where(y > 0, y, jnp.exp(jnp.minimum(y, 0.0)) - 1.0)


def _upd_body(p0_ref, p1_ref, x_ref, root_ref, bias_ref, out_ref,
              *, append_ones):
    y = (p0_ref[...] + p1_ref[...]
         + jnp.dot(x_ref[...], root_ref[...],
                   preferred_element_type=jnp.float32)
         + bias_ref[...])
    hout = _elu(y)
    if append_ones:
        out_ref[...] = jnp.concatenate(
            [hout, jnp.ones((hout.shape[0], PW - 64), jnp.float32)], axis=1)
    else:
        out_ref[...] = hout


def _upd_call(p0, p1, x, root, biasr, m_in, m_out, append_ones):
    grid = (pl.cdiv(N, TN),)
    w_out = PW if append_ones else m_out
    return pl.pallas_call(
        functools.partial(_upd_body, append_ones=append_ones),
        grid=grid,
        in_specs=[
            pl.BlockSpec((TN, m_out), lambda i: (i, 0)),
            pl.BlockSpec((TN, m_out), lambda i: (i, 0)),
            pl.BlockSpec((TN, m_in), lambda i: (i, 0)),
            pl.BlockSpec((m_in, m_out), lambda i: (0, 0)),
            pl.BlockSpec((1, m_out), lambda i: (0, 0)),
        ],
        out_specs=pl.BlockSpec((TN, w_out), lambda i: (i, 0)),
        out_shape=jax.ShapeDtypeStruct((N, w_out), jnp.float32),
    )(p0, p1, x, root, biasr)


def _final_body(p0_ref, p1_ref, s2g_ref, fc1w_ref, fc1b_ref, fc2w_ref,
                fc2b_ref, fc3w_ref, fc3b_ref, out_ref):
    tot = p0_ref[...] + p1_ref[...]
    cnt = jnp.maximum(tot[:, 64:65], 1.0)
    mean1 = tot[:, :64] / cnt
    gids = s2g_ref[...]
    onehot = jnp.where(
        lax.broadcasted_iota(jnp.int32, (G, S), 0) == gids, 1.0, 0.0)
    sums2 = jnp.dot(onehot, mean1, preferred_element_type=jnp.float32)
    cnt2 = jnp.maximum(jnp.sum(onehot, axis=1, keepdims=True), 1.0)
    mean2 = sums2 / cnt2
    a = _elu(jnp.dot(mean2, fc1w_ref[...],
                     preferred_element_type=jnp.float32) + fc1b_ref[...])
    b = _elu(jnp.dot(a, fc2w_ref[...],
                     preferred_element_type=jnp.float32) + fc2b_ref[...])
    out_ref[...] = (jnp.dot(b, fc3w_ref[...],
                            preferred_element_type=jnp.float32)
                    + fc3b_ref[...])


def _final_call(p0, p1, s2g2d, fc1w, fc1b, fc2w, fc2b, fc3w, fc3b):
    return pl.pallas_call(
        _final_body,
        out_shape=jax.ShapeDtypeStruct((G, 1), jnp.float32),
    )(p0, p1, s2g2d, fc1w, fc1b, fc2w, fc2b, fc3w, fc3b)


def kernel(x, edge_index, edge_attr, node_to_subgraph, subgraph_to_graph,
           nn1_W1, nn1_b1, nn1_W2, nn1_b2, root1, bias1,
           nn2_W1, nn2_b1, nn2_W2, nn2_b2, root2, bias2,
           nn3_W1, nn3_b1, nn3_W2, nn3_b2, root3, bias3,
           fc1_W, fc1_b, fc2_W, fc2_b, fc3_W, fc3_b):
    f32 = jnp.float32
    i32 = jnp.int32
    pad = EP - E

    def _tile_t(v, f):
        return v.reshape(EP // TE, f, TE // f).transpose(0, 2, 1).reshape(
            NW, NCH, CH)

    src_pad = jnp.concatenate(
        [edge_index[0], jnp.zeros((pad,), edge_index.dtype)]).astype(i32)
    dst_pad = jnp.concatenate(
        [edge_index[1], jnp.zeros((pad,), edge_index.dtype)]).astype(i32)
    ea_p = jnp.concatenate(
        [edge_attr.astype(f32), jnp.zeros((pad, EA), f32)])
    n2s3 = node_to_subgraph.astype(i32).reshape(NW, PNCH, PCH)
    s2g2d = subgraph_to_graph.astype(i32).reshape(1, S)

    zN64 = jnp.zeros((N, 64), f32)
    zS = jnp.zeros((S, PW), f32)

    layers = (
        (D, 32, nn1_W1, nn1_b1, nn1_W2, nn1_b2, root1, bias1),
        (32, 64, nn2_W1, nn2_b1, nn2_W2, nn2_b2, root2, bias2),
        (64, 64, nn3_W1, nn3_b1, nn3_W2, nn3_b2, root3, bias3),
    )
    h = x.astype(f32)
    for li, (m_in, m_out, w1, b1, w2, b2, root, bias) in enumerate(layers):
        xs = _make_gather(m_in)(h, _tile_t(src_pad, 128 // m_in))
        msg = _msg_call(ea_p, xs.reshape(-1, 128), w1.astype(f32),
                        b1.reshape(1, H), w2.astype(jnp.bfloat16),
                        b2.reshape(m_in, m_out), m_in, m_out)
        p = _make_scatter(m_out)(zN64[:, :m_out], msg.reshape(EP, m_out),
                                 _tile_t(dst_pad, 128 // m_out))
        h = _upd_call(p[:N], p[N:], h, root.astype(f32),
                      bias.reshape(1, m_out), m_in, m_out,
                      append_ones=(li == 2))

    pp = _make_pool()(zS, h, n2s3)
    out = _final_call(pp[:S], pp[S:], s2g2d,
                      fc1_W.astype(f32), fc1_b.reshape(1, 32),
                      fc2_W.astype(f32), fc2_b.reshape(1, 16),
                      fc3_W.astype(f32), fc3_b.reshape(1, 1))
    return out.reshape(-1)


# speedup vs baseline: 2.1604x; 1.1957x over previous
import functools

import jax
import jax.numpy as jnp
from jax import lax
from jax.experimental import pallas as pl
from jax.experimental.pallas import tpu as pltpu
from jax.experimental.pallas import tpu_sc as plsc

N = 20000
E = 60000
D = 16
S = 2000
G = 64
H = 128
EA = 5

NC = 2
NS = 16
NW = NC * NS
CH = 118
NCH = 16
EPW = NCH * CH
EP = NW * EPW

NPW = N // NW
PCH = 125
PNCH = 5
NPT = N // NS
SPT = S // NS
PW = 80

TE = 256
TN = 256


def _mesh():
    return plsc.VectorSubcoreMesh(
        core_axis_name="c", subcore_axis_name="s",
        num_cores=NC, num_subcores=NS)


@functools.lru_cache(maxsize=None)
def _make_gather(d):
    @functools.partial(
        pl.kernel,
        mesh=_mesh(),
        compiler_params=pltpu.CompilerParams(use_tc_tiling_on_sc=False),
        out_type=jax.ShapeDtypeStruct((EP, d), jnp.float32),
        scratch_types=[
            pltpu.VMEM((NCH, CH), jnp.int32),
            pltpu.VMEM((EPW, d), jnp.float32),
            pltpu.SemaphoreType.DMA,
        ],
    )
    def gather_k(table, idx, out, idx_v, rows_v, sem):
        wid = lax.axis_index("c") * NS + lax.axis_index("s")
        pltpu.sync_copy(idx.at[wid], idx_v)
        copies = [
            pltpu.async_copy(
                table.at[idx_v.at[j]], rows_v.at[pl.ds(j * CH, CH)], sem
            )
            for j in range(NCH)
        ]
        for cp in copies:
            cp.wait()
        pltpu.sync_copy(rows_v, out.at[pl.ds(wid * EPW, EPW)])

    return gather_k


@functools.lru_cache(maxsize=None)
def _make_scatter(d):
    @functools.partial(
        pl.kernel,
        mesh=_mesh(),
        compiler_params=pltpu.CompilerParams(use_tc_tiling_on_sc=False),
        out_type=jax.ShapeDtypeStruct((NC * N, d), jnp.float32),
        scratch_types=[
            pltpu.VMEM((NCH, CH), jnp.int32),
            pltpu.VMEM((2, CH, d), jnp.float32),
            pltpu.VMEM_SHARED((N, d), jnp.float32),
            pltpu.SemaphoreType.DMA,
        ],
    )
    def scatter_k(zeros, msg, idx, out, idx_v, buf, acc, sem):
        c = lax.axis_index("c")
        s = lax.axis_index("s")
        wid = c * NS + s
        pltpu.sync_copy(zeros.at[pl.ds(s * NPT, NPT)], acc.at[pl.ds(s * NPT, NPT)])
        pltpu.sync_copy(idx.at[wid], idx_v)
        plsc.subcore_barrier()
        prev = pltpu.async_copy(
            msg.at[pl.ds(wid * EPW, CH)], buf.at[0], sem)
        for j in range(NCH):
            if j + 1 < NCH:
                nxt = pltpu.async_copy(
                    msg.at[pl.ds(wid * EPW + (j + 1) * CH, CH)],
                    buf.at[(j + 1) % 2], sem)
            prev.wait()
            pltpu.sync_copy(buf.at[j % 2], acc.at[idx_v.at[j]], add=True)
            if j + 1 < NCH:
                prev = nxt
        plsc.subcore_barrier()
        pltpu.sync_copy(
            acc.at[pl.ds(s * NPT, NPT)], out.at[pl.ds(c * N + s * NPT, NPT)]
        )

    return scatter_k


@functools.lru_cache(maxsize=None)
def _make_pool():
    @functools.partial(
        pl.kernel,
        mesh=_mesh(),
        compiler_params=pltpu.CompilerParams(use_tc_tiling_on_sc=False),
        out_type=[
            jax.ShapeDtypeStruct((NC * S, 64), jnp.float32),
            jax.ShapeDtypeStruct((NC * S, 64), jnp.float32),
        ],
        scratch_types=[
            pltpu.VMEM((PNCH, PCH), jnp.int32),
            pltpu.VMEM((NPW, 64), jnp.float32),
            pltpu.VMEM((PCH, 64), jnp.float32),
            pltpu.VMEM_SHARED((S, 64), jnp.float32),
            pltpu.VMEM_SHARED((S, 64), jnp.float32),
            pltpu.SemaphoreType.DMA,
        ],
    )
    def pool_k(zeros, rows, idx, ones_h, out_h, out_c,
               idx_v, rows_v, ones_v, acc_h, acc_c, sem):
        c = lax.axis_index("c")
        s = lax.axis_index("s")
        wid = c * NS + s
        pltpu.sync_copy(zeros.at[pl.ds(s * SPT, SPT)],
                        acc_h.at[pl.ds(s * SPT, SPT)])
        pltpu.sync_copy(zeros.at[pl.ds(s * SPT, SPT)],
                        acc_c.at[pl.ds(s * SPT, SPT)])
        pltpu.sync_copy(idx.at[wid], idx_v)
        pltpu.sync_copy(rows.at[pl.ds(wid * NPW, NPW)], rows_v)
        pltpu.sync_copy(ones_h, ones_v)
        plsc.subcore_barrier()
        for j in range(PNCH):
            pltpu.sync_copy(
                rows_v.at[pl.ds(j * PCH, PCH)], acc_h.at[idx_v.at[j]],
                add=True)
            pltpu.sync_copy(ones_v, acc_c.at[idx_v.at[j]], add=True)
        plsc.subcore_barrier()
        pltpu.sync_copy(
            acc_h.at[pl.ds(s * SPT, SPT)],
            out_h.at[pl.ds(c * S + s * SPT, SPT)])
        pltpu.sync_copy(
            acc_c.at[pl.ds(s * SPT, SPT)],
            out_c.at[pl.ds(c * S + s * SPT, SPT)])

    return pool_k


def _msg_body(ea_ref, xs_ref, w1_ref, b1_ref, w2_ref, b2r_ref, r_ref, rt_ref,
              out_ref, *, m_in, m_out):
    i = pl.program_id(0)
    fi = 128 // m_in
    fo = 128 // m_out
    ea = ea_ref[...]
    h = jnp.maximum(
        jnp.dot(ea, w1_ref[...], preferred_element_type=jnp.float32)
        + b1_ref[...],
        0.0,
    )
    wt = jnp.dot(h.astype(jnp.bfloat16), w2_ref[...],
                 preferred_element_type=jnp.float32)
    x128 = xs_ref[...]
    xs = jnp.concatenate(
        [x128[:, j * m_in:(j + 1) * m_in] for j in range(fi)], axis=0)
    xb = jnp.dot(xs.astype(jnp.bfloat16), r_ref[...],
                 preferred_element_type=jnp.float32)
    prod = (xb * wt).astype(jnp.bfloat16)
    acc = jnp.dot(prod, rt_ref[...], preferred_element_type=jnp.float32)
    acc = acc + jnp.dot(xs, b2r_ref[...], preferred_element_type=jnp.float32)
    row = i * TE + lax.broadcasted_iota(jnp.int32, (TE, 1), 0)
    acc = jnp.where(row < E, acc, 0.0)
    out_ref[...] = jnp.concatenate(
        [acc[j * (TE // fo):(j + 1) * (TE // fo), :] for j in range(fo)],
        axis=1)


def _msg_call(ea_p, xs_pk, w1, b1r, w2b, b2r, m_in, m_out):
    grid = (EP // TE,)
    kk = jnp.arange(m_in * m_out, dtype=jnp.int32) // m_out
    oo = jnp.arange(m_in * m_out, dtype=jnp.int32) % m_out
    r = (jnp.arange(m_in, dtype=jnp.int32)[:, None] == kk[None, :]
         ).astype(jnp.bfloat16)
    rt = (oo[:, None] == jnp.arange(m_out, dtype=jnp.int32)[None, :]
          ).astype(jnp.bfloat16)
    fi = 128 // m_in
    fo = 128 // m_out
    return pl.pallas_call(
        functools.partial(_msg_body, m_in=m_in, m_out=m_out),
        grid=grid,
        in_specs=[
            pl.BlockSpec((TE, EA), lambda i: (i, 0)),
            pl.BlockSpec((TE // fi, 128), lambda i: (i, 0)),
            pl.BlockSpec((EA, H), lambda i: (0, 0)),
            pl.BlockSpec((1, H), lambda i: (0, 0)),
            pl.BlockSpec((H, m_in * m_out), lambda i: (0, 0)),
            pl.BlockSpec((m_in, m_out), lambda i: (0, 0)),
            pl.BlockSpec((m_in, m_in * m_out), lambda i: (0, 0)),
            pl.BlockSpec((m_in * m_out, m_out), lambda i: (0, 0)),
        ],
        out_specs=pl.BlockSpec((TE // fo, 128), lambda i: (i, 0)),
        out_shape=jax.ShapeDtypeStruct((EP // fo, 128), jnp.float32),
    )(ea_p, xs_pk, w1, b1r, w2b, b2r, r, rt)


def _elu(y):
    return jnp.where(y > 0, y, jnp.exp(jnp.minimum(y, 0.0)) - 1.0)


def _upd_body(p0_ref, p1_ref, xg_ref, rootbd_ref, biasbd_ref, out_ref):
    y = (p0_ref[...] + p1_ref[...]
         + jnp.dot(xg_ref[...], rootbd_ref[...],
                   preferred_element_type=jnp.float32)
         + biasbd_ref[...])
    out_ref[...] = _elu(y)


TB = 1000


def _upd_call(pv, xg, root, bias, m_in, m_out):
    fo = 128 // m_out
    nr = N // fo
    nb = nr // TB
    rootbd = jnp.kron(jnp.eye(fo, dtype=jnp.float32), root)
    biasbd = jnp.tile(bias, (1, fo))
    return pl.pallas_call(
        _upd_body,
        grid=(nb,),
        in_specs=[
            pl.BlockSpec((TB, 128), lambda i: (i, 0)),
            pl.BlockSpec((TB, 128), lambda i: (i + nb, 0)),
            pl.BlockSpec((TB, fo * m_in), lambda i: (i, 0)),
            pl.BlockSpec((fo * m_in, 128), lambda i: (0, 0)),
            pl.BlockSpec((1, 128), lambda i: (0, 0)),
        ],
        out_specs=pl.BlockSpec((TB, 128), lambda i: (i, 0)),
        out_shape=jax.ShapeDtypeStruct((nr, 128), jnp.float32),
    )(pv, pv, xg, rootbd, biasbd)


def _final_body(hp_ref, cp_ref, s2ge_ref, s2go_ref, fc1w_ref, fc1b_ref,
                fc2w_ref, fc2b_ref, fc3w_ref, fc3b_ref, out_ref):
    hp = hp_ref[...]
    cp = cp_ref[...]
    half = S // 2
    tot = hp[:half] + hp[half:]
    cnt = jnp.maximum(cp[:half] + cp[half:], 1.0)
    mean1 = tot / cnt
    ge = s2ge_ref[...]
    go = s2go_ref[...]
    rows = lax.broadcasted_iota(jnp.int32, (G, half), 0)
    ohe = jnp.where(rows == ge, 1.0, 0.0)
    oho = jnp.where(rows == go, 1.0, 0.0)
    sums2 = (jnp.dot(ohe, mean1[:, :64], preferred_element_type=jnp.float32)
             + jnp.dot(oho, mean1[:, 64:],
                       preferred_element_type=jnp.float32))
    cnt2 = jnp.maximum(
        jnp.sum(ohe, axis=1, keepdims=True)
        + jnp.sum(oho, axis=1, keepdims=True), 1.0)
    mean2 = sums2 / cnt2
    a = _elu(jnp.dot(mean2, fc1w_ref[...],
                     preferred_element_type=jnp.float32) + fc1b_ref[...])
    b = _elu(jnp.dot(a, fc2w_ref[...],
                     preferred_element_type=jnp.float32) + fc2b_ref[...])
    out_ref[...] = (jnp.dot(b, fc3w_ref[...],
                            preferred_element_type=jnp.float32)
                    + fc3b_ref[...])


def _final_call(hp, cp, s2ge, s2go, fc1w, fc1b, fc2w, fc2b, fc3w, fc3b):
    return pl.pallas_call(
        _final_body,
        out_shape=jax.ShapeDtypeStruct((G, 1), jnp.float32),
    )(hp, cp, s2ge, s2go, fc1w, fc1b, fc2w, fc2b, fc3w, fc3b)


def kernel(x, edge_index, edge_attr, node_to_subgraph, subgraph_to_graph,
           nn1_W1, nn1_b1, nn1_W2, nn1_b2, root1, bias1,
           nn2_W1, nn2_b1, nn2_W2, nn2_b2, root2, bias2,
           nn3_W1, nn3_b1, nn3_W2, nn3_b2, root3, bias3,
           fc1_W, fc1_b, fc2_W, fc2_b, fc3_W, fc3_b):
    f32 = jnp.float32
    i32 = jnp.int32
    pad = EP - E

    def _tile_t(v, f):
        return v.reshape(EP // TE, f, TE // f).transpose(0, 2, 1).reshape(
            NW, NCH, CH)

    src_pad = jnp.concatenate(
        [edge_index[0], jnp.zeros((pad,), edge_index.dtype)]).astype(i32)
    dst_pad = jnp.concatenate(
        [edge_index[1], jnp.zeros((pad,), edge_index.dtype)]).astype(i32)
    ea_p = jnp.concatenate(
        [edge_attr.astype(f32), jnp.zeros((pad, EA), f32)])
    n2s3 = node_to_subgraph.astype(i32).reshape(NW, PNCH, PCH)
    s2g = subgraph_to_graph.astype(i32)
    s2ge = s2g[0::2].reshape(1, S // 2)
    s2go = s2g[1::2].reshape(1, S // 2)

    zN64 = jnp.zeros((N, 64), f32)
    zS64 = jnp.zeros((S, 64), f32)
    ones_h = jnp.ones((PCH, 64), f32)

    layers = (
        (D, 32, nn1_W1, nn1_b1, nn1_W2, nn1_b2, root1, bias1),
        (32, 64, nn2_W1, nn2_b1, nn2_W2, nn2_b2, root2, bias2),
        (64, 64, nn3_W1, nn3_b1, nn3_W2, nn3_b2, root3, bias3),
    )
    h = x.astype(f32)
    for li, (m_in, m_out, w1, b1, w2, b2, root, bias) in enumerate(layers):
        xs = _make_gather(m_in)(h, _tile_t(src_pad, 128 // m_in))
        msg = _msg_call(ea_p, xs.reshape(-1, 128), w1.astype(f32),
                        b1.reshape(1, H), w2.astype(jnp.bfloat16),
                        b2.reshape(m_in, m_out), m_in, m_out)
        p = _make_scatter(m_out)(zN64[:, :m_out], msg.reshape(EP, m_out),
                                 _tile_t(dst_pad, 128 // m_out))
        fo = 128 // m_out
        hp = _upd_call(p.reshape(-1, 128), h.reshape(N // fo, fo * m_in),
                       root.astype(f32), bias.reshape(1, m_out), m_in, m_out)
        h = hp.reshape(N, m_out)

    pp_h, pp_c = _make_pool()(zS64, h, n2s3, ones_h)
    out = _final_call(pp_h.reshape(S, 128), pp_c.reshape(S, 128),
                      s2ge, s2go,
                      fc1_W.astype(f32), fc1_b.reshape(1, 32),
                      fc2_W.astype(f32), fc2_b.reshape(1, 16),
                      fc3_W.astype(f32), fc3_b.reshape(1, 1))
    return out.reshape(-1)


# speedup vs baseline: 2.1907x; 1.0140x over previous
import functools

import jax
import jax.numpy as jnp
from jax import lax
from jax.experimental import pallas as pl
from jax.experimental.pallas import tpu as pltpu
from jax.experimental.pallas import tpu_sc as plsc

N = 20000
E = 60000
D = 16
S = 2000
G = 64
H = 128
EA = 5

NC = 2
NS = 16
NW = NC * NS
CH = 118
NCH = 16
EPW = NCH * CH
EP = NW * EPW

NPW = N // NW
PCH = 125
PNCH = 5
NPT = N // NS
SPT = S // NS
PW = 80

TE = 512
TN = 256


def _mesh():
    return plsc.VectorSubcoreMesh(
        core_axis_name="c", subcore_axis_name="s",
        num_cores=NC, num_subcores=NS)


@functools.lru_cache(maxsize=None)
def _make_gather(d):
    @functools.partial(
        pl.kernel,
        mesh=_mesh(),
        compiler_params=pltpu.CompilerParams(use_tc_tiling_on_sc=False),
        out_type=jax.ShapeDtypeStruct((EP, d), jnp.float32),
        scratch_types=[
            pltpu.VMEM((NCH, CH), jnp.int32),
            pltpu.VMEM((EPW, d), jnp.float32),
            pltpu.SemaphoreType.DMA,
        ],
    )
    def gather_k(table, idx, out, idx_v, rows_v, sem):
        wid = lax.axis_index("c") * NS + lax.axis_index("s")
        pltpu.sync_copy(idx.at[wid], idx_v)
        copies = [
            pltpu.async_copy(
                table.at[idx_v.at[j]], rows_v.at[pl.ds(j * CH, CH)], sem
            )
            for j in range(NCH)
        ]
        for cp in copies:
            cp.wait()
        pltpu.sync_copy(rows_v, out.at[pl.ds(wid * EPW, EPW)])

    return gather_k


@functools.lru_cache(maxsize=None)
def _make_scatter(d):
    @functools.partial(
        pl.kernel,
        mesh=_mesh(),
        compiler_params=pltpu.CompilerParams(use_tc_tiling_on_sc=False),
        out_type=jax.ShapeDtypeStruct((NC * N, d), jnp.float32),
        scratch_types=[
            pltpu.VMEM((NCH, CH), jnp.int32),
            pltpu.VMEM((2, CH, d), jnp.float32),
            pltpu.VMEM_SHARED((N, d), jnp.float32),
            pltpu.SemaphoreType.DMA,
        ],
    )
    def scatter_k(zeros, msg, idx, out, idx_v, buf, acc, sem):
        c = lax.axis_index("c")
        s = lax.axis_index("s")
        wid = c * NS + s
        pltpu.sync_copy(zeros.at[pl.ds(s * NPT, NPT)], acc.at[pl.ds(s * NPT, NPT)])
        pltpu.sync_copy(idx.at[wid], idx_v)
        plsc.subcore_barrier()
        prev = pltpu.async_copy(
            msg.at[pl.ds(wid * EPW, CH)], buf.at[0], sem)
        for j in range(NCH):
            if j + 1 < NCH:
                nxt = pltpu.async_copy(
                    msg.at[pl.ds(wid * EPW + (j + 1) * CH, CH)],
                    buf.at[(j + 1) % 2], sem)
            prev.wait()
            pltpu.sync_copy(buf.at[j % 2], acc.at[idx_v.at[j]], add=True)
            if j + 1 < NCH:
                prev = nxt
        plsc.subcore_barrier()
        pltpu.sync_copy(
            acc.at[pl.ds(s * NPT, NPT)], out.at[pl.ds(c * N + s * NPT, NPT)]
        )

    return scatter_k


@functools.lru_cache(maxsize=None)
def _make_pool():
    @functools.partial(
        pl.kernel,
        mesh=_mesh(),
        compiler_params=pltpu.CompilerParams(use_tc_tiling_on_sc=False),
        out_type=[
            jax.ShapeDtypeStruct((NC * S, 64), jnp.float32),
            jax.ShapeDtypeStruct((NC * S, 64), jnp.float32),
        ],
        scratch_types=[
            pltpu.VMEM((PNCH, PCH), jnp.int32),
            pltpu.VMEM((NPW, 64), jnp.float32),
            pltpu.VMEM((PCH, 64), jnp.float32),
            pltpu.VMEM_SHARED((S, 64), jnp.float32),
            pltpu.VMEM_SHARED((S, 64), jnp.float32),
            pltpu.SemaphoreType.DMA,
        ],
    )
    def pool_k(zeros, rows, idx, ones_h, out_h, out_c,
               idx_v, rows_v, ones_v, acc_h, acc_c, sem):
        c = lax.axis_index("c")
        s = lax.axis_index("s")
        wid = c * NS + s
        pltpu.sync_copy(zeros.at[pl.ds(s * SPT, SPT)],
                        acc_h.at[pl.ds(s * SPT, SPT)])
        pltpu.sync_copy(zeros.at[pl.ds(s * SPT, SPT)],
                        acc_c.at[pl.ds(s * SPT, SPT)])
        pltpu.sync_copy(idx.at[wid], idx_v)
        pltpu.sync_copy(rows.at[pl.ds(wid * NPW, NPW)], rows_v)
        pltpu.sync_copy(ones_h, ones_v)
        plsc.subcore_barrier()
        for j in range(PNCH):
            pltpu.sync_copy(
                rows_v.at[pl.ds(j * PCH, PCH)], acc_h.at[idx_v.at[j]],
                add=True)
            pltpu.sync_copy(ones_v, acc_c.at[idx_v.at[j]], add=True)
        plsc.subcore_barrier()
        pltpu.sync_copy(
            acc_h.at[pl.ds(s * SPT, SPT)],
            out_h.at[pl.ds(c * S + s * SPT, SPT)])
        pltpu.sync_copy(
            acc_c.at[pl.ds(s * SPT, SPT)],
            out_c.at[pl.ds(c * S + s * SPT, SPT)])

    return pool_k


def _msg_body(ea_ref, xs_ref, w1_ref, b1_ref, w2_ref, b2r_ref, r_ref, rt_ref,
              out_ref, *, m_in, m_out):
    i = pl.program_id(0)
    fi = 128 // m_in
    fo = 128 // m_out
    ea = ea_ref[...]
    h = jnp.maximum(
        jnp.dot(ea, w1_ref[...], preferred_element_type=jnp.float32)
        + b1_ref[...],
        0.0,
    )
    wt = jnp.dot(h.astype(jnp.bfloat16), w2_ref[...],
                 preferred_element_type=jnp.float32)
    x128 = xs_ref[...]
    xs = jnp.concatenate(
        [x128[:, j * m_in:(j + 1) * m_in] for j in range(fi)], axis=0)
    xb = jnp.dot(xs.astype(jnp.bfloat16), r_ref[...],
                 preferred_element_type=jnp.float32)
    prod = (xb * wt).astype(jnp.bfloat16)
    acc = jnp.dot(prod, rt_ref[...], preferred_element_type=jnp.float32)
    acc = acc + jnp.dot(xs, b2r_ref[...], preferred_element_type=jnp.float32)
    row = i * TE + lax.broadcasted_iota(jnp.int32, (TE, 1), 0)
    acc = jnp.where(row < E, acc, 0.0)
    out_ref[...] = jnp.concatenate(
        [acc[j * (TE // fo):(j + 1) * (TE // fo), :] for j in range(fo)],
        axis=1)


def _msg_call(ea_p, xs_pk, w1, b1r, w2b, b2r, m_in, m_out):
    grid = (EP // TE,)
    kk = jnp.arange(m_in * m_out, dtype=jnp.int32) // m_out
    oo = jnp.arange(m_in * m_out, dtype=jnp.int32) % m_out
    r = (jnp.arange(m_in, dtype=jnp.int32)[:, None] == kk[None, :]
         ).astype(jnp.bfloat16)
    rt = (oo[:, None] == jnp.arange(m_out, dtype=jnp.int32)[None, :]
          ).astype(jnp.bfloat16)
    fi = 128 // m_in
    fo = 128 // m_out
    return pl.pallas_call(
        functools.partial(_msg_body, m_in=m_in, m_out=m_out),
        grid=grid,
        in_specs=[
            pl.BlockSpec((TE, EA), lambda i: (i, 0)),
            pl.BlockSpec((TE // fi, 128), lambda i: (i, 0)),
            pl.BlockSpec((EA, H), lambda i: (0, 0)),
            pl.BlockSpec((1, H), lambda i: (0, 0)),
            pl.BlockSpec((H, m_in * m_out), lambda i: (0, 0)),
            pl.BlockSpec((m_in, m_out), lambda i: (0, 0)),
            pl.BlockSpec((m_in, m_in * m_out), lambda i: (0, 0)),
            pl.BlockSpec((m_in * m_out, m_out), lambda i: (0, 0)),
        ],
        out_specs=pl.BlockSpec((TE // fo, 128), lambda i: (i, 0)),
        out_shape=jax.ShapeDtypeStruct((EP // fo, 128), jnp.float32),
    )(ea_p, xs_pk, w1, b1r, w2b, b2r, r, rt)


def _elu(y):
    return jnp.where(y > 0, y, jnp.exp(jnp.minimum(y, 0.0)) - 1.0)


def _upd_body(p0_ref, p1_ref, xg_ref, rootbd_ref, biasbd_ref, out_ref):
    y = (p0_ref[...] + p1_ref[...]
         + jnp.dot(xg_ref[...], rootbd_ref[...],
                   preferred_element_type=jnp.float32)
         + biasbd_ref[...])
    out_ref[...] = _elu(y)


TB = 1000


def _upd_call(pv, xg, root, bias, m_in, m_out):
    fo = 128 // m_out
    nr = N // fo
    nb = nr // TB
    rootbd = jnp.kron(jnp.eye(fo, dtype=jnp.float32), root)
    biasbd = jnp.tile(bias, (1, fo))
    return pl.pallas_call(
        _upd_body,
        grid=(nb,),
        in_specs=[
            pl.BlockSpec((TB, 128), lambda i: (i, 0)),
            pl.BlockSpec((TB, 128), lambda i: (i + nb, 0)),
            pl.BlockSpec((TB, fo * m_in), lambda i: (i, 0)),
            pl.BlockSpec((fo * m_in, 128), lambda i: (0, 0)),
            pl.BlockSpec((1, 128), lambda i: (0, 0)),
        ],
        out_specs=pl.BlockSpec((TB, 128), lambda i: (i, 0)),
        out_shape=jax.ShapeDtypeStruct((nr, 128), jnp.float32),
    )(pv, pv, xg, rootbd, biasbd)


def _final_body(hp_ref, cp_ref, s2ge_ref, s2go_ref, fc1w_ref, fc1b_ref,
                fc2w_ref, fc2b_ref, fc3w_ref, fc3b_ref, out_ref):
    hp = hp_ref[...]
    cp = cp_ref[...]
    half = S // 2
    tot = hp[:half] + hp[half:]
    cnt = jnp.maximum(cp[:half] + cp[half:], 1.0)
    mean1 = tot / cnt
    ge = s2ge_ref[...]
    go = s2go_ref[...]
    rows = lax.broadcasted_iota(jnp.int32, (G, half), 0)
    ohe = jnp.where(rows == ge, 1.0, 0.0)
    oho = jnp.where(rows == go, 1.0, 0.0)
    sums2 = (jnp.dot(ohe, mean1[:, :64], preferred_element_type=jnp.float32)
             + jnp.dot(oho, mean1[:, 64:],
                       preferred_element_type=jnp.float32))
    cnt2 = jnp.maximum(
        jnp.sum(ohe, axis=1, keepdims=True)
        + jnp.sum(oho, axis=1, keepdims=True), 1.0)
    mean2 = sums2 / cnt2
    a = _elu(jnp.dot(mean2, fc1w_ref[...],
                     preferred_element_type=jnp.float32) + fc1b_ref[...])
    b = _elu(jnp.dot(a, fc2w_ref[...],
                     preferred_element_type=jnp.float32) + fc2b_ref[...])
    out_ref[...] = (jnp.dot(b, fc3w_ref[...],
                            preferred_element_type=jnp.float32)
                    + fc3b_ref[...])


def _final_call(hp, cp, s2ge, s2go, fc1w, fc1b, fc2w, fc2b, fc3w, fc3b):
    return pl.pallas_call(
        _final_body,
        out_shape=jax.ShapeDtypeStruct((G, 1), jnp.float32),
    )(hp, cp, s2ge, s2go, fc1w, fc1b, fc2w, fc2b, fc3w, fc3b)


def kernel(x, edge_index, edge_attr, node_to_subgraph, subgraph_to_graph,
           nn1_W1, nn1_b1, nn1_W2, nn1_b2, root1, bias1,
           nn2_W1, nn2_b1, nn2_W2, nn2_b2, root2, bias2,
           nn3_W1, nn3_b1, nn3_W2, nn3_b2, root3, bias3,
           fc1_W, fc1_b, fc2_W, fc2_b, fc3_W, fc3_b):
    f32 = jnp.float32
    i32 = jnp.int32
    pad = EP - E

    def _tile_t(v, f):
        return v.reshape(EP // TE, f, TE // f).transpose(0, 2, 1).reshape(
            NW, NCH, CH)

    src_pad = jnp.concatenate(
        [edge_index[0], jnp.zeros((pad,), edge_index.dtype)]).astype(i32)
    dst_pad = jnp.concatenate(
        [edge_index[1], jnp.zeros((pad,), edge_index.dtype)]).astype(i32)
    ea_p = jnp.concatenate(
        [edge_attr.astype(f32), jnp.zeros((pad, EA), f32)])
    n2s3 = node_to_subgraph.astype(i32).reshape(NW, PNCH, PCH)
    s2g = subgraph_to_graph.astype(i32)
    s2ge = s2g[0::2].reshape(1, S // 2)
    s2go = s2g[1::2].reshape(1, S // 2)

    zN64 = jnp.zeros((N, 64), f32)
    zS64 = jnp.zeros((S, 64), f32)
    ones_h = jnp.ones((PCH, 64), f32)

    layers = (
        (D, 32, nn1_W1, nn1_b1, nn1_W2, nn1_b2, root1, bias1),
        (32, 64, nn2_W1, nn2_b1, nn2_W2, nn2_b2, root2, bias2),
        (64, 64, nn3_W1, nn3_b1, nn3_W2, nn3_b2, root3, bias3),
    )
    h = x.astype(f32)
    for li, (m_in, m_out, w1, b1, w2, b2, root, bias) in enumerate(layers):
        xs = _make_gather(m_in)(h, _tile_t(src_pad, 128 // m_in))
        msg = _msg_call(ea_p, xs.reshape(-1, 128), w1.astype(f32),
                        b1.reshape(1, H), w2.astype(jnp.bfloat16),
                        b2.reshape(m_in, m_out), m_in, m_out)
        p = _make_scatter(m_out)(zN64[:, :m_out], msg.reshape(EP, m_out),
                                 _tile_t(dst_pad, 128 // m_out))
        fo = 128 // m_out
        hp = _upd_call(p.reshape(-1, 128), h.reshape(N // fo, fo * m_in),
                       root.astype(f32), bias.reshape(1, m_out), m_in, m_out)
        h = hp.reshape(N, m_out)

    pp_h, pp_c = _make_pool()(zS64, h, n2s3, ones_h)
    out = _final_call(pp_h.reshape(S, 128), pp_c.reshape(S, 128),
                      s2ge, s2go,
                      fc1_W.astype(f32), fc1_b.reshape(1, 32),
                      fc2_W.astype(f32), fc2_b.reshape(1, 16),
                      fc3_W.astype(f32), fc3_b.reshape(1, 1))
    return out.reshape(-1)


# speedup vs baseline: 2.3768x; 1.0849x over previous
import functools

import jax
import jax.numpy as jnp
from jax import lax
from jax.experimental import pallas as pl
from jax.experimental.pallas import tpu as pltpu
from jax.experimental.pallas import tpu_sc as plsc

N = 20000
E = 60000
D = 16
S = 2000
G = 64
H = 128
EA = 5

NC = 2
NS = 16
NW = NC * NS
CH = 118
NCH = 16
EPW = NCH * CH
EP = NW * EPW

NPW = N // NW
PCH = 125
PNCH = 5
NPT = N // NS
SPT = S // NS
PW = 80

TE = 1024
TN = 256


def _mesh():
    return plsc.VectorSubcoreMesh(
        core_axis_name="c", subcore_axis_name="s",
        num_cores=NC, num_subcores=NS)


@functools.lru_cache(maxsize=None)
def _make_gather(d):
    @functools.partial(
        pl.kernel,
        mesh=_mesh(),
        compiler_params=pltpu.CompilerParams(use_tc_tiling_on_sc=False),
        out_type=jax.ShapeDtypeStruct((EP, d), jnp.float32),
        scratch_types=[
            pltpu.VMEM((NCH, CH), jnp.int32),
            pltpu.VMEM((EPW, d), jnp.float32),
            pltpu.SemaphoreType.DMA,
        ],
    )
    def gather_k(table, idx, out, idx_v, rows_v, sem):
        wid = lax.axis_index("c") * NS + lax.axis_index("s")
        pltpu.sync_copy(idx.at[wid], idx_v)
        copies = [
            pltpu.async_copy(
                table.at[idx_v.at[j]], rows_v.at[pl.ds(j * CH, CH)], sem
            )
            for j in range(NCH)
        ]
        for cp in copies:
            cp.wait()
        pltpu.sync_copy(rows_v, out.at[pl.ds(wid * EPW, EPW)])

    return gather_k


@functools.lru_cache(maxsize=None)
def _make_scatter(d):
    @functools.partial(
        pl.kernel,
        mesh=_mesh(),
        compiler_params=pltpu.CompilerParams(use_tc_tiling_on_sc=False),
        out_type=jax.ShapeDtypeStruct((NC * N, d), jnp.float32),
        scratch_types=[
            pltpu.VMEM((NCH, CH), jnp.int32),
            pltpu.VMEM((2, CH, d), jnp.float32),
            pltpu.VMEM_SHARED((N, d), jnp.float32),
            pltpu.SemaphoreType.DMA,
        ],
    )
    def scatter_k(zeros, msg, idx, out, idx_v, buf, acc, sem):
        c = lax.axis_index("c")
        s = lax.axis_index("s")
        wid = c * NS + s
        pltpu.sync_copy(zeros.at[pl.ds(s * NPT, NPT)], acc.at[pl.ds(s * NPT, NPT)])
        pltpu.sync_copy(idx.at[wid], idx_v)
        plsc.subcore_barrier()
        prev = pltpu.async_copy(
            msg.at[pl.ds(wid * EPW, CH)], buf.at[0], sem)
        for j in range(NCH):
            if j + 1 < NCH:
                nxt = pltpu.async_copy(
                    msg.at[pl.ds(wid * EPW + (j + 1) * CH, CH)],
                    buf.at[(j + 1) % 2], sem)
            prev.wait()
            pltpu.sync_copy(buf.at[j % 2], acc.at[idx_v.at[j]], add=True)
            if j + 1 < NCH:
                prev = nxt
        plsc.subcore_barrier()
        pltpu.sync_copy(
            acc.at[pl.ds(s * NPT, NPT)], out.at[pl.ds(c * N + s * NPT, NPT)]
        )

    return scatter_k


@functools.lru_cache(maxsize=None)
def _make_pool():
    @functools.partial(
        pl.kernel,
        mesh=_mesh(),
        compiler_params=pltpu.CompilerParams(use_tc_tiling_on_sc=False),
        out_type=[
            jax.ShapeDtypeStruct((NC * S, 64), jnp.float32),
            jax.ShapeDtypeStruct((NC * S, 64), jnp.float32),
        ],
        scratch_types=[
            pltpu.VMEM((PNCH, PCH), jnp.int32),
            pltpu.VMEM((NPW, 64), jnp.float32),
            pltpu.VMEM((PCH, 64), jnp.float32),
            pltpu.VMEM_SHARED((S, 64), jnp.float32),
            pltpu.VMEM_SHARED((S, 64), jnp.float32),
            pltpu.SemaphoreType.DMA,
        ],
    )
    def pool_k(zeros, rows, idx, ones_h, out_h, out_c,
               idx_v, rows_v, ones_v, acc_h, acc_c, sem):
        c = lax.axis_index("c")
        s = lax.axis_index("s")
        wid = c * NS + s
        pltpu.sync_copy(zeros.at[pl.ds(s * SPT, SPT)],
                        acc_h.at[pl.ds(s * SPT, SPT)])
        pltpu.sync_copy(zeros.at[pl.ds(s * SPT, SPT)],
                        acc_c.at[pl.ds(s * SPT, SPT)])
        pltpu.sync_copy(idx.at[wid], idx_v)
        pltpu.sync_copy(rows.at[pl.ds(wid * NPW, NPW)], rows_v)
        pltpu.sync_copy(ones_h, ones_v)
        plsc.subcore_barrier()
        for j in range(PNCH):
            pltpu.sync_copy(
                rows_v.at[pl.ds(j * PCH, PCH)], acc_h.at[idx_v.at[j]],
                add=True)
            pltpu.sync_copy(ones_v, acc_c.at[idx_v.at[j]], add=True)
        plsc.subcore_barrier()
        pltpu.sync_copy(
            acc_h.at[pl.ds(s * SPT, SPT)],
            out_h.at[pl.ds(c * S + s * SPT, SPT)])
        pltpu.sync_copy(
            acc_c.at[pl.ds(s * SPT, SPT)],
            out_c.at[pl.ds(c * S + s * SPT, SPT)])

    return pool_k


def _msg_body(ea_ref, xs_ref, w1_ref, b1_ref, w2_ref, b2r_ref, r_ref, rt_ref,
              out_ref, *, m_in, m_out):
    i = pl.program_id(0)
    fi = 128 // m_in
    fo = 128 // m_out
    ea = ea_ref[...]
    h = jnp.maximum(
        jnp.dot(ea, w1_ref[...], preferred_element_type=jnp.float32)
        + b1_ref[...],
        0.0,
    )
    wt = jnp.dot(h.astype(jnp.bfloat16), w2_ref[...],
                 preferred_element_type=jnp.float32)
    x128 = xs_ref[...]
    xs = jnp.concatenate(
        [x128[:, j * m_in:(j + 1) * m_in] for j in range(fi)], axis=0)
    xb = jnp.dot(xs.astype(jnp.bfloat16), r_ref[...],
                 preferred_element_type=jnp.float32)
    prod = (xb * wt).astype(jnp.bfloat16)
    acc = jnp.dot(prod, rt_ref[...], preferred_element_type=jnp.float32)
    acc = acc + jnp.dot(xs, b2r_ref[...], preferred_element_type=jnp.float32)
    row = i * TE + lax.broadcasted_iota(jnp.int32, (TE, 1), 0)
    acc = jnp.where(row < E, acc, 0.0)
    out_ref[...] = jnp.concatenate(
        [acc[j * (TE // fo):(j + 1) * (TE // fo), :] for j in range(fo)],
        axis=1)


def _msg_call(ea_p, xs_pk, w1, b1r, w2b, b2r, m_in, m_out):
    grid = (EP // TE,)
    kk = jnp.arange(m_in * m_out, dtype=jnp.int32) // m_out
    oo = jnp.arange(m_in * m_out, dtype=jnp.int32) % m_out
    r = (jnp.arange(m_in, dtype=jnp.int32)[:, None] == kk[None, :]
         ).astype(jnp.bfloat16)
    rt = (oo[:, None] == jnp.arange(m_out, dtype=jnp.int32)[None, :]
          ).astype(jnp.bfloat16)
    fi = 128 // m_in
    fo = 128 // m_out
    return pl.pallas_call(
        functools.partial(_msg_body, m_in=m_in, m_out=m_out),
        grid=grid,
        in_specs=[
            pl.BlockSpec((TE, EA), lambda i: (i, 0)),
            pl.BlockSpec((TE // fi, 128), lambda i: (i, 0)),
            pl.BlockSpec((EA, H), lambda i: (0, 0)),
            pl.BlockSpec((1, H), lambda i: (0, 0)),
            pl.BlockSpec((H, m_in * m_out), lambda i: (0, 0)),
            pl.BlockSpec((m_in, m_out), lambda i: (0, 0)),
            pl.BlockSpec((m_in, m_in * m_out), lambda i: (0, 0)),
            pl.BlockSpec((m_in * m_out, m_out), lambda i: (0, 0)),
        ],
        out_specs=pl.BlockSpec((TE // fo, 128), lambda i: (i, 0)),
        out_shape=jax.ShapeDtypeStruct((EP // fo, 128), jnp.float32),
    )(ea_p, xs_pk, w1, b1r, w2b, b2r, r, rt)


def _elu(y):
    return jnp.where(y > 0, y, jnp.exp(jnp.minimum(y, 0.0)) - 1.0)


def _upd_body(p0_ref, p1_ref, xg_ref, rootbd_ref, biasbd_ref, out_ref):
    y = (p0_ref[...] + p1_ref[...]
         + jnp.dot(xg_ref[...], rootbd_ref[...],
                   preferred_element_type=jnp.float32)
         + biasbd_ref[...])
    out_ref[...] = _elu(y)


TB = 1000


def _upd_call(pv, xg, root, bias, m_in, m_out):
    fo = 128 // m_out
    nr = N // fo
    nb = nr // TB
    rootbd = jnp.kron(jnp.eye(fo, dtype=jnp.float32), root)
    biasbd = jnp.tile(bias, (1, fo))
    return pl.pallas_call(
        _upd_body,
        grid=(nb,),
        in_specs=[
            pl.BlockSpec((TB, 128), lambda i: (i, 0)),
            pl.BlockSpec((TB, 128), lambda i: (i + nb, 0)),
            pl.BlockSpec((TB, fo * m_in), lambda i: (i, 0)),
            pl.BlockSpec((fo * m_in, 128), lambda i: (0, 0)),
            pl.BlockSpec((1, 128), lambda i: (0, 0)),
        ],
        out_specs=pl.BlockSpec((TB, 128), lambda i: (i, 0)),
        out_shape=jax.ShapeDtypeStruct((nr, 128), jnp.float32),
    )(pv, pv, xg, rootbd, biasbd)


def _final_body(hp_ref, cp_ref, s2ge_ref, s2go_ref, fc1w_ref, fc1b_ref,
                fc2w_ref, fc2b_ref, fc3w_ref, fc3b_ref, out_ref):
    hp = hp_ref[...]
    cp = cp_ref[...]
    half = S // 2
    tot = hp[:half] + hp[half:]
    cnt = jnp.maximum(cp[:half] + cp[half:], 1.0)
    mean1 = tot / cnt
    ge = s2ge_ref[...]
    go = s2go_ref[...]
    rows = lax.broadcasted_iota(jnp.int32, (G, half), 0)
    ohe = jnp.where(rows == ge, 1.0, 0.0)
    oho = jnp.where(rows == go, 1.0, 0.0)
    sums2 = (jnp.dot(ohe, mean1[:, :64], preferred_element_type=jnp.float32)
             + jnp.dot(oho, mean1[:, 64:],
                       preferred_element_type=jnp.float32))
    cnt2 = jnp.maximum(
        jnp.sum(ohe, axis=1, keepdims=True)
        + jnp.sum(oho, axis=1, keepdims=True), 1.0)
    mean2 = sums2 / cnt2
    a = _elu(jnp.dot(mean2, fc1w_ref[...],
                     preferred_element_type=jnp.float32) + fc1b_ref[...])
    b = _elu(jnp.dot(a, fc2w_ref[...],
                     preferred_element_type=jnp.float32) + fc2b_ref[...])
    out_ref[...] = (jnp.dot(b, fc3w_ref[...],
                            preferred_element_type=jnp.float32)
                    + fc3b_ref[...])


def _final_call(hp, cp, s2ge, s2go, fc1w, fc1b, fc2w, fc2b, fc3w, fc3b):
    return pl.pallas_call(
        _final_body,
        out_shape=jax.ShapeDtypeStruct((G, 1), jnp.float32),
    )(hp, cp, s2ge, s2go, fc1w, fc1b, fc2w, fc2b, fc3w, fc3b)


def kernel(x, edge_index, edge_attr, node_to_subgraph, subgraph_to_graph,
           nn1_W1, nn1_b1, nn1_W2, nn1_b2, root1, bias1,
           nn2_W1, nn2_b1, nn2_W2, nn2_b2, root2, bias2,
           nn3_W1, nn3_b1, nn3_W2, nn3_b2, root3, bias3,
           fc1_W, fc1_b, fc2_W, fc2_b, fc3_W, fc3_b):
    f32 = jnp.float32
    i32 = jnp.int32
    pad = EP - E

    def _tile_t(v, f):
        return v.reshape(EP // TE, f, TE // f).transpose(0, 2, 1).reshape(
            NW, NCH, CH)

    src_pad = jnp.concatenate(
        [edge_index[0], jnp.zeros((pad,), edge_index.dtype)]).astype(i32)
    dst_pad = jnp.concatenate(
        [edge_index[1], jnp.zeros((pad,), edge_index.dtype)]).astype(i32)
    ea_p = jnp.concatenate(
        [edge_attr.astype(f32), jnp.zeros((pad, EA), f32)])
    n2s3 = node_to_subgraph.astype(i32).reshape(NW, PNCH, PCH)
    s2g = subgraph_to_graph.astype(i32)
    s2ge = s2g[0::2].reshape(1, S // 2)
    s2go = s2g[1::2].reshape(1, S // 2)

    zN64 = jnp.zeros((N, 64), f32)
    zS64 = jnp.zeros((S, 64), f32)
    ones_h = jnp.ones((PCH, 64), f32)

    layers = (
        (D, 32, nn1_W1, nn1_b1, nn1_W2, nn1_b2, root1, bias1),
        (32, 64, nn2_W1, nn2_b1, nn2_W2, nn2_b2, root2, bias2),
        (64, 64, nn3_W1, nn3_b1, nn3_W2, nn3_b2, root3, bias3),
    )
    h = x.astype(f32)
    for li, (m_in, m_out, w1, b1, w2, b2, root, bias) in enumerate(layers):
        xs = _make_gather(m_in)(h, _tile_t(src_pad, 128 // m_in))
        msg = _msg_call(ea_p, xs.reshape(-1, 128), w1.astype(f32),
                        b1.reshape(1, H), w2.astype(jnp.bfloat16),
                        b2.reshape(m_in, m_out), m_in, m_out)
        p = _make_scatter(m_out)(zN64[:, :m_out], msg.reshape(EP, m_out),
                                 _tile_t(dst_pad, 128 // m_out))
        fo = 128 // m_out
        hp = _upd_call(p.reshape(-1, 128), h.reshape(N // fo, fo * m_in),
                       root.astype(f32), bias.reshape(1, m_out), m_in, m_out)
        h = hp.reshape(N, m_out)

    pp_h, pp_c = _make_pool()(zS64, h, n2s3, ones_h)
    out = _final_call(pp_h.reshape(S, 128), pp_c.reshape(S, 128),
                      s2ge, s2go,
                      fc1_W.astype(f32), fc1_b.reshape(1, 32),
                      fc2_W.astype(f32), fc2_b.reshape(1, 16),
                      fc3_W.astype(f32), fc3_b.reshape(1, 1))
    return out.reshape(-1)
